# baseline (speedup 1.0000x reference)
import jax
import jax.numpy as jnp
from jax import lax
from jax.experimental import pallas as pl
from jax.experimental.pallas import tpu as pltpu
from jax.experimental.pallas import tpu_sc as plsc

N = 10000
E = 160000
D = 128
ED = 16
H = 4
C = 64
T = 8
GINE = 64
OUT = 128

NC = 2
NS = 16
NW = NC * NS
EK = 128
NCHUNK = E // EK
MAXCH = -(-NCHUNK // NW)
NP = 10240
RPT = NP // NS
ZCHUNKS = ((0, 128), (128, 128), (256, 128), (384, 128), (512, 128))
EXS = E * H // NC

_BN = 1000
_BE = 2000


def _mesh():
    return plsc.VectorSubcoreMesh(
        core_axis_name="c", subcore_axis_name="s", num_cores=NC, num_subcores=NS)


def _tca_body(x_ref, wg_ref, asdw_ref, r0, r1, r2, r3, x0, x1, asd_ref):
    xb = x_ref[...]
    h = jnp.dot(xb, wg_ref[...], preferred_element_type=jnp.float32)
    r0[...] = h[:, 0:64]
    r1[...] = h[:, 64:128]
    r2[...] = h[:, 128:192]
    r3[...] = h[:, 192:256]
    x0[...] = xb[:, 0:64]
    x1[...] = xb[:, 64:128]
    asd_ref[...] = jnp.dot(h, asdw_ref[...], preferred_element_type=jnp.float32)


def _tca(x, W_gat, Asd):
    q = lambda: pl.BlockSpec((_BN, 64), lambda i: (i, 0))
    return pl.pallas_call(
        _tca_body,
        grid=(N // _BN,),
        in_specs=[
            pl.BlockSpec((_BN, D), lambda i: (i, 0)),
            pl.BlockSpec((D, H * C), lambda i: (0, 0)),
            pl.BlockSpec((H * C, 8), lambda i: (0, 0)),
        ],
        out_specs=[q(), q(), q(), q(), q(), q(),
                   pl.BlockSpec((_BN, 8), lambda i: (i, 0))],
        out_shape=[jax.ShapeDtypeStruct((N, 64), jnp.float32)] * 6
        + [jax.ShapeDtypeStruct((N, 8), jnp.float32)],
    )(x, W_gat, Asd)


def _tcb_body(ea_ref, eaf_ref, et_ref, w0_ref, w1_ref, aep_ref, tg_ref,
              epp0_ref, epp1_ref, ae8_ref):
    ea = ea_ref[...]
    epp0_ref[...] = jnp.dot(ea, w0_ref[...], preferred_element_type=jnp.float32)
    epp1_ref[...] = jnp.dot(ea, w1_ref[...], preferred_element_type=jnp.float32)
    oh = (lax.broadcasted_iota(jnp.int32, (_BE, T), 1) == et_ref[...]).astype(jnp.float32)
    ae8_ref[...] = (
        jnp.dot(eaf_ref[...], aep_ref[...], preferred_element_type=jnp.float32)
        + jnp.dot(oh, tg_ref[...], preferred_element_type=jnp.float32))


def _tcb(ea32, edge_attr, et2d, W0p, W1p, AEP8, tG8):
    return pl.pallas_call(
        _tcb_body,
        grid=(E // _BE,),
        in_specs=[
            pl.BlockSpec((_BE // 2, 2 * ED), lambda i: (i, 0)),
            pl.BlockSpec((_BE, ED), lambda i: (i, 0)),
            pl.BlockSpec((_BE, 1), lambda i: (i, 0)),
            pl.BlockSpec((2 * ED, D), lambda i: (0, 0)),
            pl.BlockSpec((2 * ED, D), lambda i: (0, 0)),
            pl.BlockSpec((ED, 8), lambda i: (0, 0)),
            pl.BlockSpec((T, 8), lambda i: (0, 0)),
        ],
        out_specs=[
            pl.BlockSpec((_BE // 2, D), lambda i: (i, 0)),
            pl.BlockSpec((_BE // 2, D), lambda i: (i, 0)),
            pl.BlockSpec((_BE, 8), lambda i: (i, 0)),
        ],
        out_shape=[
            jax.ShapeDtypeStruct((E // 2, D), jnp.float32),
            jax.ShapeDtypeStruct((E // 2, D), jnp.float32),
            jax.ShapeDtypeStruct((E, 8), jnp.float32),
        ],
    )(ea32, edge_attr, et2d, W0p, W1p, AEP8, tG8)


def _zero_vmem_rows(ref, nrows, ncols):
    z16 = jnp.zeros((16,), jnp.float32)

    def body(r, _):
        for cb in range(ncols // 16):
            ref[r, pl.ds(cb * 16, 16)] = z16
        return 0

    lax.fori_loop(0, nrows, body, 0, unroll=False)


def _zero_my_shared_rows(zbuf, shared, base):
    for off, cnt in ZCHUNKS:
        pltpu.sync_copy(zbuf.at[pl.ds(0, cnt)], shared.at[pl.ds(base + off, cnt)])


def _dump_my_shared_rows(shared, out, cid, base):
    for off, cnt in ZCHUNKS:
        pltpu.sync_copy(shared.at[pl.ds(base + off, cnt)],
                        out.at[cid, pl.ds(base + off, cnt)])


def _compute_ex(asrc_v, adst_v, ae_v, ex4_v, ex8_v, iota16):
    for j16 in range(8):
        rows = iota16 + (j16 * 16)
        for j in range(H):
            colj = jnp.full((16,), j, jnp.int32)
            a_s = plsc.load_gather(asrc_v, [rows, colj])
            a_d = plsc.load_gather(adst_v, [rows, colj + 4])
            a_e = plsc.load_gather(ae_v, [rows, colj])
            lg = a_s + a_d + a_e
            lg = jnp.maximum(lg, lg * 0.2)
            exv = jnp.exp(lg)
            plsc.store_scatter(ex4_v, [rows * 4 + j], exv)
            plsc.store_scatter(ex8_v, [rows, colj], exv)


def _scale_quarter(t_v, ex4_v, head):
    def body(e, _):
        ef = jnp.full((16,), e * 4 + head, jnp.int32)
        b = plsc.load_gather(ex4_v, [ef])
        for cb in range(4):
            sl = pl.ds(cb * 16, 16)
            t_v[e, sl] = t_v[e, sl] * b
        return 0

    lax.fori_loop(0, EK, body, 0, unroll=False)


def _relu_add_quarter(t_v, epp_v, et_i, teh_v, iota16):
    for g in range(8):
        rows = iota16 + g * 16
        et16 = et_i[pl.ds(g * 16, 16)]
        er = rows // 2
        eo = (rows & 1) * 64

        def cbody(col, _):
            cv = jnp.full((16,), col, jnp.int32)
            tev = jnp.zeros((16,), jnp.float32)
            cr = jnp.full((16,), col // 8, jnp.int32)
            cc = jnp.full((16,), col % 8, jnp.int32)
            for t in range(T):
                wt = plsc.load_gather(teh_v, [cr + t * 8, cc])
                tev = jnp.where(et16 == float(t), wt, tev)
            v = (plsc.load_gather(t_v, [rows, cv])
                 + plsc.load_gather(epp_v, [er, eo + col])
                 + tev)
            plsc.store_scatter(t_v, [rows, cv], jnp.maximum(v, 0.0))
            return 0

        lax.fori_loop(0, 64, cbody, 0, unroll=False)


def _sc_body(r0, r1, r2, r3, x0, x1, asd, ae8, etH, epp0, epp1, srcH, dstH,
             q0_out, q1_out, q2_out, q3_out, g0_out, g1_out, denom_out,
             src_a, src_b, dst_a, dst_b, ea_a, ea_b, et_a, et_b,
             asrc_a, asrc_b, adst_a, adst_b, ex4_v, ex8_v, t_a, t_b,
             epp_a, epp_b, aep_v, tg_v, te0_v, te1_v,
             sem_a, sem_b, ex_sp, denom_sh, acc_sh):
    cid = lax.axis_index("c")
    sid = lax.axis_index("s")
    wid = sid * NC + cid
    base = sid * RPT

    srcb = (src_a, src_b)
    dstb = (dst_a, dst_b)
    eab = (ea_a, ea_b)
    etb = (et_a, et_b)
    asrcb = (asrc_a, asrc_b)
    adstb = (adst_a, adst_b)
    tb = (t_a, t_b)
    eppb = (epp_a, epp_b)
    semb = (sem_a, sem_b)

    iota16 = lax.iota(jnp.int32, 16)

    pltpu.sync_copy(asd.at[pl.ds(N, ED)], aep_v)
    pltpu.sync_copy(asd.at[pl.ds(N + 16, T)], tg_v)
    pltpu.sync_copy(asd.at[pl.ds(N + 24, 64)], te0_v)
    pltpu.sync_copy(asd.at[pl.ds(N + 88, 64)], te1_v)

    _zero_vmem_rows(t_a, EK, 64)

    def zex(g, _):
        rows = iota16 // 8 + 2 * g
        cols = jnp.bitwise_and(iota16, 7)
        plsc.store_scatter(ex8_v, [rows, cols], jnp.zeros((16,), jnp.float32))
        return 0

    lax.fori_loop(0, EK // 2, zex, 0, unroll=False)
    _zero_my_shared_rows(t_a, acc_sh, base)
    _zero_my_shared_rows(ex8_v, denom_sh, base)
    plsc.subcore_barrier()

    def run_stage(table, stage, head, epph, teh_v, out):
        def prefetch(nb, c):
            e0 = c * EK
            pltpu.sync_copy(srcH.at[pl.ds(e0, EK)], srcb[nb])
            pltpu.sync_copy(dstH.at[pl.ds(e0, EK)], dstb[nb])
            pltpu.async_copy(table.at[srcb[nb]], tb[nb], semb[nb])
            if stage == 0:
                pltpu.sync_copy(ae8.at[pl.ds(e0, EK)], eab[nb])
                pltpu.async_copy(asd.at[srcb[nb]], asrcb[nb], semb[nb])
                pltpu.async_copy(asd.at[dstb[nb]], adstb[nb], semb[nb])
            if epph is not None:
                pltpu.sync_copy(etH.at[pl.ds(e0, EK)], etb[nb])
                pltpu.async_copy(
                    epph.at[pl.ds(e0 // 2, EK // 2)], eppb[nb], semb[nb])

        def wait(b):
            pltpu.make_async_copy(table.at[srcb[b]], tb[b], semb[b]).wait()
            if stage == 0:
                pltpu.make_async_copy(asd.at[srcb[b]], asrcb[b], semb[b]).wait()
                pltpu.make_async_copy(asd.at[dstb[b]], adstb[b], semb[b]).wait()
            if epph is not None:
                pltpu.make_async_copy(
                    epph.at[pl.ds(0, EK // 2)], eppb[b], semb[b]).wait()

        def compute(b, c):
            slot = (c // NC) * (EK * H)
            if stage == 0:
                _compute_ex(asrcb[b], adstb[b], eab[b], ex4_v, ex8_v, iota16)
                pltpu.sync_copy(ex4_v, ex_sp.at[pl.ds(slot, EK * H)])
                pltpu.sync_copy(ex8_v, denom_sh.at[dstb[b]], add=True)
            elif head is not None:
                pltpu.sync_copy(ex_sp.at[pl.ds(slot, EK * H)], ex4_v)
            if head is not None:
                _scale_quarter(tb[b], ex4_v, head)
            else:
                _relu_add_quarter(tb[b], eppb[b], etb[b], teh_v, iota16)
            pltpu.sync_copy(tb[b], acc_sh.at[dstb[b]], add=True)

        prefetch(0, wid)

        def iter_k(k, _):
            for bb in range(2):
                i = 2 * k + bb
                c = wid + i * NW
                nc = c + NW

                @pl.when(nc < NCHUNK)
                def _():
                    prefetch(1 - bb, nc)

                @pl.when(c < NCHUNK)
                def _():
                    wait(bb)
                    compute(bb, c)
            return 0

        lax.fori_loop(0, MAXCH // 2, iter_k, 0, unroll=False)
        plsc.subcore_barrier()
        _dump_my_shared_rows(acc_sh, out, cid, base)
        if stage == 0:
            _dump_my_shared_rows(denom_sh, denom_out, cid, base)
        if stage < 5:
            _zero_vmem_rows(t_a, EK, 64)
            _zero_my_shared_rows(t_a, acc_sh, base)
        plsc.subcore_barrier()

    run_stage(r0, 0, 0, None, None, q0_out)
    run_stage(r1, 1, 1, None, None, q1_out)
    run_stage(r2, 2, 2, None, None, q2_out)
    run_stage(r3, 3, 3, None, None, q3_out)
    run_stage(x0, 4, None, epp0, te0_v, g0_out)
    run_stage(x1, 5, None, epp1, te1_v, g1_out)


def _sc(r0, r1, r2, r3, x0, x1, asd, ae8, et, epp0, epp1, src, dst):
    qo = lambda: jax.ShapeDtypeStruct((NC, NP, 64), jnp.float32)
    return pl.kernel(
        _sc_body,
        out_type=[qo(), qo(), qo(), qo(), qo(), qo(),
                  jax.ShapeDtypeStruct((NC, NP, 8), jnp.float32)],
        mesh=_mesh(),
        compiler_params=pltpu.CompilerParams(
            use_tc_tiling_on_sc=False, needs_layout_passes=False),
        scratch_types=[
            pltpu.VMEM((EK,), jnp.int32),
            pltpu.VMEM((EK,), jnp.int32),
            pltpu.VMEM((EK,), jnp.int32),
            pltpu.VMEM((EK,), jnp.int32),
            pltpu.VMEM((EK, 8), jnp.float32),
            pltpu.VMEM((EK, 8), jnp.float32),
            pltpu.VMEM((EK,), jnp.float32),
            pltpu.VMEM((EK,), jnp.float32),
            pltpu.VMEM((EK, 8), jnp.float32),
            pltpu.VMEM((EK, 8), jnp.float32),
            pltpu.VMEM((EK, 8), jnp.float32),
            pltpu.VMEM((EK, 8), jnp.float32),
            pltpu.VMEM((EK * H,), jnp.float32),
            pltpu.VMEM((EK, 8), jnp.float32),
            pltpu.VMEM((EK, 64), jnp.float32),
            pltpu.VMEM((EK, 64), jnp.float32),
            pltpu.VMEM((EK // 2, D), jnp.float32),
            pltpu.VMEM((EK // 2, D), jnp.float32),
            pltpu.VMEM((ED, 8), jnp.float32),
            pltpu.VMEM((T, 8), jnp.float32),
            pltpu.VMEM((64, 8), jnp.float32),
            pltpu.VMEM((64, 8), jnp.float32),
            pltpu.SemaphoreType.DMA,
            pltpu.SemaphoreType.DMA,
            pltpu.VMEM_SHARED((EXS,), jnp.float32),
            pltpu.VMEM_SHARED((NP, 8), jnp.float32),
            pltpu.VMEM_SHARED((NP, 64), jnp.float32),
        ],
    )(r0, r1, r2, r3, x0, x1, asd, ae8, et, epp0, epp1, src, dst)


def _tcc_body(x_ref, q0_ref, q1_ref, q2_ref, q3_ref, g0_ref, g1_ref, d_ref,
              one64_ref, w1a_ref, w1b_ref, b1_ref, w2_ref, b2_ref,
              cw0_ref, cw1_ref, cw2_ref, cw3_ref, cwb_ref, zb_ref,
              lg_ref, lb_ref, out_ref):
    den = d_ref[0, :, :4] + d_ref[1, :, :4]
    dinv = 1.0 / (den + 1e-16)
    one64 = one64_ref[...]
    qs = (q0_ref, q1_ref, q2_ref, q3_ref)
    cws = (cw0_ref, cw1_ref, cw2_ref, cw3_ref)
    z = jnp.broadcast_to(zb_ref[...], (_BN, OUT))
    for h in range(H):
        s = jnp.dot(dinv[:, h:h + 1], one64, preferred_element_type=jnp.float32)
        num = qs[h][0] + qs[h][1]
        z = z + jnp.dot(num * s, cws[h][...], preferred_element_type=jnp.float32)
    xb = x_ref[...]
    hg0 = xb[:, 0:64] + g0_ref[0] + g0_ref[1]
    hg1 = xb[:, 64:128] + g1_ref[0] + g1_ref[1]
    t = jnp.maximum(
        jnp.dot(hg0, w1a_ref[...], preferred_element_type=jnp.float32)
        + jnp.dot(hg1, w1b_ref[...], preferred_element_type=jnp.float32)
        + b1_ref[...], 0.0)
    g = jnp.dot(t, w2_ref[...], preferred_element_type=jnp.float32) + b2_ref[...]
    z = z + jnp.dot(g, cwb_ref[...], preferred_element_type=jnp.float32)
    mu = jnp.mean(z, axis=-1, keepdims=True)
    zc = z - mu
    var = jnp.mean(zc * zc, axis=-1, keepdims=True)
    zn = zc * lax.rsqrt(var + 1e-5) * lg_ref[...] + lb_ref[...]
    out_ref[...] = jnp.maximum(zn, 0.0)


def _tcc(x, qs, gs, denom_p, one64, w1a, w1b, mlp_b1, mlp_w2, mlp_b2,
         cw, cwb, zb, ln_gamma, ln_beta):
    full = lambda *shape: pl.BlockSpec(shape, lambda i: (0,) * len(shape))
    pq = lambda: pl.BlockSpec((NC, _BN, 64), lambda i: (0, i, 0))
    return pl.pallas_call(
        _tcc_body,
        grid=(N // _BN,),
        in_specs=[
            pl.BlockSpec((_BN, D), lambda i: (i, 0)),
            pq(), pq(), pq(), pq(), pq(), pq(),
            pl.BlockSpec((NC, _BN, 8), lambda i: (0, i, 0)),
            full(1, 64),
            full(64, GINE),
            full(64, GINE),
            full(1, GINE),
            full(GINE, GINE),
            full(1, GINE),
            full(64, OUT),
            full(64, OUT),
            full(64, OUT),
            full(64, OUT),
            full(GINE, OUT),
            full(1, OUT),
            full(1, OUT),
            full(1, OUT),
        ],
        out_specs=pl.BlockSpec((_BN, OUT), lambda i: (i, 0)),
        out_shape=jax.ShapeDtypeStruct((N, OUT), jnp.float32),
    )(x, qs[0], qs[1], qs[2], qs[3], gs[0], gs[1], denom_p, one64,
      w1a, w1b, mlp_b1, mlp_w2, mlp_b2, cw[0], cw[1], cw[2], cw[3], cwb,
      zb, ln_gamma, ln_beta)


def kernel(x, edge_index, edge_attr, edge_types, type_emb_gat, W_gat,
           W_edge_gat, att_src, att_dst, att_edge, bias_gat, type_emb_gine,
           edge_lin_w, edge_lin_b, mlp_w1, mlp_b1, mlp_w2, mlp_b2, comb_w,
           comb_b, ln_gamma, ln_beta):
    src = edge_index[0].astype(jnp.int32)
    dst = edge_index[1].astype(jnp.int32)
    et = edge_types.astype(jnp.float32)
    ea32 = edge_attr.reshape(E // 2, 2 * ED)

    ar = jnp.arange(H)
    Asrc = jnp.zeros((H, C, H), jnp.float32).at[ar, :, ar].set(att_src)
    Adst = jnp.zeros((H, C, H), jnp.float32).at[ar, :, ar].set(att_dst)
    Asd = jnp.concatenate(
        [Asrc.reshape(H * C, H), Adst.reshape(H * C, H)], axis=1)
    AEP = jnp.einsum("ehc,hc->eh", W_edge_gat.reshape(ED, H, C), att_edge)
    AEP8 = jnp.pad(AEP, ((0, 0), (0, 4)))
    tG8 = jnp.dot(type_emb_gat, AEP8)
    tE = jnp.dot(type_emb_gine, edge_lin_w) + edge_lin_b[None]
    tE0 = tE[:, 0:64]
    tE1 = tE[:, 64:128]
    W0p = jnp.zeros((2 * ED, D), jnp.float32)
    W0p = W0p.at[0:ED, 0:64].set(edge_lin_w[:, 0:64])
    W0p = W0p.at[ED:2 * ED, 64:128].set(edge_lin_w[:, 0:64])
    W1p = jnp.zeros((2 * ED, D), jnp.float32)
    W1p = W1p.at[0:ED, 0:64].set(edge_lin_w[:, 64:128])
    W1p = W1p.at[ED:2 * ED, 64:128].set(edge_lin_w[:, 64:128])
    one64 = jnp.ones((1, 64), jnp.float32)
    cw = [comb_w[64 * i:64 * (i + 1)] for i in range(4)]
    cwb = comb_w[256:]
    zb = (comb_b + jnp.dot(bias_gat, comb_w[:256]))[None]
    w1a = mlp_w1[:64]
    w1b = mlp_w1[64:]

    r0, r1, r2, r3, x0, x1, asd = _tca(x, W_gat, Asd)
    wt = jnp.concatenate(
        [AEP8, tG8, tE0.reshape(64, 8), tE1.reshape(64, 8)], axis=0)
    asd_ext = jnp.concatenate([asd, wt], axis=0)
    et2d = edge_types.astype(jnp.int32).reshape(E, 1)
    epp0, epp1, ae8 = _tcb(ea32, edge_attr, et2d, W0p, W1p, AEP8, tG8)
    q0, q1, q2, q3, g0, g1, denom_p = _sc(
        r0, r1, r2, r3, x0, x1, asd_ext, ae8, et, epp0, epp1, src, dst)
    return _tcc(x, (q0, q1, q2, q3), (g0, g1), denom_p, one64, w1a, w1b,
                mlp_b1.reshape(1, GINE), mlp_w2, mlp_b2.reshape(1, GINE),
                cw, cwb, zb, ln_gamma.reshape(1, OUT), ln_beta.reshape(1, OUT))


# speedup vs baseline: 1.9975x; 1.9975x over previous
import jax
import jax.numpy as jnp
from jax import lax
from jax.experimental import pallas as pl
from jax.experimental.pallas import tpu as pltpu
from jax.experimental.pallas import tpu_sc as plsc

N = 10000
E = 160000
D = 128
ED = 16
H = 4
C = 64
T = 8
GINE = 64
OUT = 128

NC = 2
NS = 16
NW = NC * NS
EK = 128
NCHUNK = E // EK
MAXCH = -(-NCHUNK // NW)
NP = 10240
RPT = NP // NS
ZCHUNKS = ((0, 128), (128, 128), (256, 128), (384, 128), (512, 128))
EXS = E * H // NC

_BN = 1000
_BE = 2000


def _mesh():
    return plsc.VectorSubcoreMesh(
        core_axis_name="c", subcore_axis_name="s", num_cores=NC, num_subcores=NS)


def _tca_body(x_ref, wg_ref, asdw_ref, r0, r1, r2, r3, x0, x1, asd_ref):
    xb = x_ref[...]
    h = jnp.dot(xb, wg_ref[...], preferred_element_type=jnp.float32)
    r0[...] = h[:, 0:64]
    r1[...] = h[:, 64:128]
    r2[...] = h[:, 128:192]
    r3[...] = h[:, 192:256]
    x0[...] = xb[:, 0:64]
    x1[...] = xb[:, 64:128]
    asd_ref[...] = jnp.dot(h, asdw_ref[...], preferred_element_type=jnp.float32)


def _tca(x, W_gat, Asd):
    q = lambda: pl.BlockSpec((_BN, 64), lambda i: (i, 0))
    return pl.pallas_call(
        _tca_body,
        grid=(N // _BN,),
        in_specs=[
            pl.BlockSpec((_BN, D), lambda i: (i, 0)),
            pl.BlockSpec((D, H * C), lambda i: (0, 0)),
            pl.BlockSpec((H * C, 8), lambda i: (0, 0)),
        ],
        out_specs=[q(), q(), q(), q(), q(), q(),
                   pl.BlockSpec((_BN, 8), lambda i: (i, 0))],
        out_shape=[jax.ShapeDtypeStruct((N, 64), jnp.float32)] * 6
        + [jax.ShapeDtypeStruct((N, 8), jnp.float32)],
    )(x, W_gat, Asd)


def _tcb_body(ea_ref, etp_ref, w0_ref, w1_ref, t0l_ref, t0r_ref, t1l_ref,
              t1r_ref, aep_ref, tgl_ref, tgr_ref, epp0_ref, epp1_ref, ae_ref):
    f32 = jnp.float32
    ea = ea_ref[...]
    etp = etp_ref[...]
    BE2 = _BE // 2
    oh_e = (lax.broadcasted_iota(jnp.int32, (BE2, T), 1) == etp[:, 0:1]).astype(f32)
    oh_o = (lax.broadcasted_iota(jnp.int32, (BE2, T), 1) == etp[:, 1:2]).astype(f32)
    dot = lambda a, b: jnp.dot(a, b, preferred_element_type=f32)
    epp0_ref[...] = dot(ea, w0_ref[...]) + dot(oh_e, t0l_ref[...]) + dot(oh_o, t0r_ref[...])
    epp1_ref[...] = dot(ea, w1_ref[...]) + dot(oh_e, t1l_ref[...]) + dot(oh_o, t1r_ref[...])
    ae_ref[...] = dot(ea, aep_ref[...]) + dot(oh_e, tgl_ref[...]) + dot(oh_o, tgr_ref[...])


def _tcb(ea32, etP2, W0p, W1p, T0L, T0R, T1L, T1R, AEPp, TGL, TGR):
    full = lambda *shape: pl.BlockSpec(shape, lambda i: (0,) * len(shape))
    return pl.pallas_call(
        _tcb_body,
        grid=(E // _BE,),
        in_specs=[
            pl.BlockSpec((_BE // 2, 2 * ED), lambda i: (i, 0)),
            pl.BlockSpec((_BE // 2, 2), lambda i: (i, 0)),
            full(2 * ED, D), full(2 * ED, D),
            full(T, D), full(T, D), full(T, D), full(T, D),
            full(2 * ED, 16), full(T, 16), full(T, 16),
        ],
        out_specs=[
            pl.BlockSpec((_BE // 2, D), lambda i: (i, 0)),
            pl.BlockSpec((_BE // 2, D), lambda i: (i, 0)),
            pl.BlockSpec((_BE // 2, 16), lambda i: (i, 0)),
        ],
        out_shape=[
            jax.ShapeDtypeStruct((E // 2, D), jnp.float32),
            jax.ShapeDtypeStruct((E // 2, D), jnp.float32),
            jax.ShapeDtypeStruct((E // 2, 16), jnp.float32),
        ],
    )(ea32, etP2, W0p, W1p, T0L, T0R, T1L, T1R, AEPp, TGL, TGR)


def _zero_vmem_rows(ref, nrows, ncols):
    z16 = jnp.zeros((16,), jnp.float32)

    def body(r, _):
        for cb in range(ncols // 16):
            ref[r, pl.ds(cb * 16, 16)] = z16
        return 0

    lax.fori_loop(0, nrows, body, 0, unroll=False)


def _zero_my_shared_rows(zbuf, shared, base):
    for off, cnt in ZCHUNKS:
        pltpu.sync_copy(zbuf.at[pl.ds(0, cnt)], shared.at[pl.ds(base + off, cnt)])


def _dump_my_shared_rows(shared, out, cid, base):
    for off, cnt in ZCHUNKS:
        pltpu.sync_copy(shared.at[pl.ds(base + off, cnt)],
                        out.at[cid, pl.ds(base + off, cnt)])


def _compute_ex(asrc_v, adst_v, ae_v, ex4_v, ex8_v, iota16):
    for j16 in range(8):
        rows = iota16 + (j16 * 16)
        for j in range(H):
            colj = jnp.full((16,), j, jnp.int32)
            a_s = plsc.load_gather(asrc_v, [rows, colj])
            a_d = plsc.load_gather(adst_v, [rows, colj + 4])
            a_e = plsc.load_gather(ae_v, [rows * 8 + j])
            lg = a_s + a_d + a_e
            lg = jnp.maximum(lg, lg * 0.2)
            exv = jnp.exp(lg)
            plsc.store_scatter(ex4_v, [rows * 4 + j], exv)
            plsc.store_scatter(ex8_v, [rows, colj], exv)


def _scale_quarter(t_v, ex4_v, head):
    def body(e, _):
        ef = jnp.full((16,), e * 4 + head, jnp.int32)
        b = plsc.load_gather(ex4_v, [ef])
        for cb in range(4):
            sl = pl.ds(cb * 16, 16)
            t_v[e, sl] = t_v[e, sl] * b
        return 0

    lax.fori_loop(0, EK, body, 0, unroll=False)


def _relu_add_quarter(t_v, epp_v, iota16):
    def body(e, _):
        for cb in range(4):
            sl = pl.ds(cb * 16, 16)
            ep = plsc.load_gather(
                epp_v, [jnp.full((16,), e * 64 + cb * 16, jnp.int32) + iota16])
            t_v[e, sl] = jnp.maximum(t_v[e, sl] + ep, 0.0)
        return 0

    lax.fori_loop(0, EK, body, 0, unroll=False)


def _sc_body(r0, r1, r2, r3, x0, x1, asd, aefl, epp0, epp1, srcH, dstH,
             q0_out, q1_out, q2_out, q3_out, g0_out, g1_out, denom_out,
             src_a, src_b, dst_a, dst_b, ea_a, ea_b,
             asrc_a, asrc_b, adst_a, adst_b, ex4_v, ex8_v, t_a, t_b,
             epp_a, epp_b, sem_a, sem_b, ex_sp, denom_sh, acc_sh):
    cid = lax.axis_index("c")
    sid = lax.axis_index("s")
    wid = sid * NC + cid
    base = sid * RPT

    srcb = (src_a, src_b)
    dstb = (dst_a, dst_b)
    eab = (ea_a, ea_b)
    asrcb = (asrc_a, asrc_b)
    adstb = (adst_a, adst_b)
    tb = (t_a, t_b)
    eppb = (epp_a, epp_b)
    semb = (sem_a, sem_b)

    iota16 = lax.iota(jnp.int32, 16)

    _zero_vmem_rows(t_a, EK, 64)

    def zex(g, _):
        rows = iota16 // 8 + 2 * g
        cols = jnp.bitwise_and(iota16, 7)
        plsc.store_scatter(ex8_v, [rows, cols], jnp.zeros((16,), jnp.float32))
        return 0

    lax.fori_loop(0, EK // 2, zex, 0, unroll=False)
    _zero_my_shared_rows(t_a, acc_sh, base)
    _zero_my_shared_rows(ex8_v, denom_sh, base)
    plsc.subcore_barrier()

    def run_stage(table, stage, head, epph, out):
        def prefetch(nb, c):
            e0 = c * EK
            pltpu.sync_copy(srcH.at[pl.ds(e0, EK)], srcb[nb])
            pltpu.sync_copy(dstH.at[pl.ds(e0, EK)], dstb[nb])
            pltpu.async_copy(table.at[srcb[nb]], tb[nb], semb[nb])
            if stage == 0:
                pltpu.sync_copy(aefl.at[pl.ds(e0 * 8, EK * 8)], eab[nb])
                pltpu.async_copy(asd.at[srcb[nb]], asrcb[nb], semb[nb])
                pltpu.async_copy(asd.at[dstb[nb]], adstb[nb], semb[nb])
            if epph is not None:
                pltpu.async_copy(
                    epph.at[pl.ds(e0 * 64, EK * 64)], eppb[nb], semb[nb])

        def wait(b):
            pltpu.make_async_copy(table.at[srcb[b]], tb[b], semb[b]).wait()
            if stage == 0:
                pltpu.make_async_copy(asd.at[srcb[b]], asrcb[b], semb[b]).wait()
                pltpu.make_async_copy(asd.at[dstb[b]], adstb[b], semb[b]).wait()
            if epph is not None:
                pltpu.make_async_copy(
                    epph.at[pl.ds(0, EK * 64)], eppb[b], semb[b]).wait()

        def compute(b, c):
            slot = (c // NC) * (EK * H)
            if stage == 0:
                _compute_ex(asrcb[b], adstb[b], eab[b], ex4_v, ex8_v, iota16)
                pltpu.sync_copy(ex4_v, ex_sp.at[pl.ds(slot, EK * H)])
                pltpu.sync_copy(ex8_v, denom_sh.at[dstb[b]], add=True)
            elif head is not None:
                pltpu.sync_copy(ex_sp.at[pl.ds(slot, EK * H)], ex4_v)
            if head is not None:
                _scale_quarter(tb[b], ex4_v, head)
            else:
                _relu_add_quarter(tb[b], eppb[b], iota16)
            pltpu.sync_copy(tb[b], acc_sh.at[dstb[b]], add=True)

        prefetch(0, wid)

        def iter_k(k, _):
            for bb in range(2):
                i = 2 * k + bb
                c = wid + i * NW
                nc = c + NW

                @pl.when(nc < NCHUNK)
                def _():
                    prefetch(1 - bb, nc)

                @pl.when(c < NCHUNK)
                def _():
                    wait(bb)
                    compute(bb, c)
            return 0

        lax.fori_loop(0, MAXCH // 2, iter_k, 0, unroll=False)
        plsc.subcore_barrier()
        _dump_my_shared_rows(acc_sh, out, cid, base)
        if stage == 0:
            _dump_my_shared_rows(denom_sh, denom_out, cid, base)
        if stage < 5:
            _zero_vmem_rows(t_a, EK, 64)
            _zero_my_shared_rows(t_a, acc_sh, base)
        plsc.subcore_barrier()

    run_stage(r0, 0, 0, None, q0_out)
    run_stage(r1, 1, 1, None, q1_out)
    run_stage(r2, 2, 2, None, q2_out)
    run_stage(r3, 3, 3, None, q3_out)
    run_stage(x0, 4, None, epp0, g0_out)
    run_stage(x1, 5, None, epp1, g1_out)


def _sc(r0, r1, r2, r3, x0, x1, asd, aefl, epp0, epp1, src, dst):
    qo = lambda: jax.ShapeDtypeStruct((NC, NP, 64), jnp.float32)
    return pl.kernel(
        _sc_body,
        out_type=[qo(), qo(), qo(), qo(), qo(), qo(),
                  jax.ShapeDtypeStruct((NC, NP, 8), jnp.float32)],
        mesh=_mesh(),
        compiler_params=pltpu.CompilerParams(
            use_tc_tiling_on_sc=False, needs_layout_passes=False),
        scratch_types=[
            pltpu.VMEM((EK,), jnp.int32),
            pltpu.VMEM((EK,), jnp.int32),
            pltpu.VMEM((EK,), jnp.int32),
            pltpu.VMEM((EK,), jnp.int32),
            pltpu.VMEM((EK * 8,), jnp.float32),
            pltpu.VMEM((EK * 8,), jnp.float32),
            pltpu.VMEM((EK, 8), jnp.float32),
            pltpu.VMEM((EK, 8), jnp.float32),
            pltpu.VMEM((EK, 8), jnp.float32),
            pltpu.VMEM((EK, 8), jnp.float32),
            pltpu.VMEM((EK * H,), jnp.float32),
            pltpu.VMEM((EK, 8), jnp.float32),
            pltpu.VMEM((EK, 64), jnp.float32),
            pltpu.VMEM((EK, 64), jnp.float32),
            pltpu.VMEM((EK * 64,), jnp.float32),
            pltpu.VMEM((EK * 64,), jnp.float32),
            pltpu.SemaphoreType.DMA,
            pltpu.SemaphoreType.DMA,
            pltpu.VMEM_SHARED((EXS,), jnp.float32),
            pltpu.VMEM_SHARED((NP, 8), jnp.float32),
            pltpu.VMEM_SHARED((NP, 64), jnp.float32),
        ],
    )(r0, r1, r2, r3, x0, x1, asd, aefl, epp0, epp1, src, dst)


def _tcc_body(x_ref, q0_ref, q1_ref, q2_ref, q3_ref, g0_ref, g1_ref, d_ref,
              one64_ref, w1a_ref, w1b_ref, b1_ref, w2_ref, b2_ref,
              cw0_ref, cw1_ref, cw2_ref, cw3_ref, cwb_ref, zb_ref,
              lg_ref, lb_ref, out_ref):
    den = d_ref[0, :, :4] + d_ref[1, :, :4]
    dinv = 1.0 / (den + 1e-16)
    one64 = one64_ref[...]
    qs = (q0_ref, q1_ref, q2_ref, q3_ref)
    cws = (cw0_ref, cw1_ref, cw2_ref, cw3_ref)
    z = jnp.broadcast_to(zb_ref[...], (_BN, OUT))
    for h in range(H):
        s = jnp.dot(dinv[:, h:h + 1], one64, preferred_element_type=jnp.float32)
        num = qs[h][0] + qs[h][1]
        z = z + jnp.dot(num * s, cws[h][...], preferred_element_type=jnp.float32)
    xb = x_ref[...]
    hg0 = xb[:, 0:64] + g0_ref[0] + g0_ref[1]
    hg1 = xb[:, 64:128] + g1_ref[0] + g1_ref[1]
    t = jnp.maximum(
        jnp.dot(hg0, w1a_ref[...], preferred_element_type=jnp.float32)
        + jnp.dot(hg1, w1b_ref[...], preferred_element_type=jnp.float32)
        + b1_ref[...], 0.0)
    g = jnp.dot(t, w2_ref[...], preferred_element_type=jnp.float32) + b2_ref[...]
    z = z + jnp.dot(g, cwb_ref[...], preferred_element_type=jnp.float32)
    mu = jnp.mean(z, axis=-1, keepdims=True)
    zc = z - mu
    var = jnp.mean(zc * zc, axis=-1, keepdims=True)
    zn = zc * lax.rsqrt(var + 1e-5) * lg_ref[...] + lb_ref[...]
    out_ref[...] = jnp.maximum(zn, 0.0)


def _tcc(x, qs, gs, denom_p, one64, w1a, w1b, mlp_b1, mlp_w2, mlp_b2,
         cw, cwb, zb, ln_gamma, ln_beta):
    full = lambda *shape: pl.BlockSpec(shape, lambda i: (0,) * len(shape))
    pq = lambda: pl.BlockSpec((NC, _BN, 64), lambda i: (0, i, 0))
    return pl.pallas_call(
        _tcc_body,
        grid=(N // _BN,),
        in_specs=[
            pl.BlockSpec((_BN, D), lambda i: (i, 0)),
            pq(), pq(), pq(), pq(), pq(), pq(),
            pl.BlockSpec((NC, _BN, 8), lambda i: (0, i, 0)),
            full(1, 64),
            full(64, GINE),
            full(64, GINE),
            full(1, GINE),
            full(GINE, GINE),
            full(1, GINE),
            full(64, OUT),
            full(64, OUT),
            full(64, OUT),
            full(64, OUT),
            full(GINE, OUT),
            full(1, OUT),
            full(1, OUT),
            full(1, OUT),
        ],
        out_specs=pl.BlockSpec((_BN, OUT), lambda i: (i, 0)),
        out_shape=jax.ShapeDtypeStruct((N, OUT), jnp.float32),
    )(x, qs[0], qs[1], qs[2], qs[3], gs[0], gs[1], denom_p, one64,
      w1a, w1b, mlp_b1, mlp_w2, mlp_b2, cw[0], cw[1], cw[2], cw[3], cwb,
      zb, ln_gamma, ln_beta)


def kernel(x, edge_index, edge_attr, edge_types, type_emb_gat, W_gat,
           W_edge_gat, att_src, att_dst, att_edge, bias_gat, type_emb_gine,
           edge_lin_w, edge_lin_b, mlp_w1, mlp_b1, mlp_w2, mlp_b2, comb_w,
           comb_b, ln_gamma, ln_beta):
    src = edge_index[0].astype(jnp.int32)
    dst = edge_index[1].astype(jnp.int32)
    et = edge_types.astype(jnp.float32)
    ea32 = edge_attr.reshape(E // 2, 2 * ED)

    f32 = jnp.float32
    ar = jnp.arange(H)
    Asrc = jnp.zeros((H, C, H), f32).at[ar, :, ar].set(att_src)
    Adst = jnp.zeros((H, C, H), f32).at[ar, :, ar].set(att_dst)
    Asd = jnp.concatenate(
        [Asrc.reshape(H * C, H), Adst.reshape(H * C, H)], axis=1)
    AEP = jnp.einsum("ehc,hc->eh", W_edge_gat.reshape(ED, H, C), att_edge)
    AEP8 = jnp.pad(AEP, ((0, 0), (0, 4)))
    tG8 = jnp.dot(type_emb_gat, AEP8)
    tE = jnp.dot(type_emb_gine, edge_lin_w) + edge_lin_b[None]
    W0p = jnp.zeros((2 * ED, D), f32)
    W0p = W0p.at[0:ED, 0:64].set(edge_lin_w[:, 0:64])
    W0p = W0p.at[ED:2 * ED, 64:128].set(edge_lin_w[:, 0:64])
    W1p = jnp.zeros((2 * ED, D), f32)
    W1p = W1p.at[0:ED, 0:64].set(edge_lin_w[:, 64:128])
    W1p = W1p.at[ED:2 * ED, 64:128].set(edge_lin_w[:, 64:128])
    T0L = jnp.zeros((T, D), f32).at[:, 0:64].set(tE[:, 0:64])
    T0R = jnp.zeros((T, D), f32).at[:, 64:128].set(tE[:, 0:64])
    T1L = jnp.zeros((T, D), f32).at[:, 0:64].set(tE[:, 64:128])
    T1R = jnp.zeros((T, D), f32).at[:, 64:128].set(tE[:, 64:128])
    AEPp = jnp.zeros((2 * ED, 16), f32)
    AEPp = AEPp.at[0:ED, 0:8].set(AEP8)
    AEPp = AEPp.at[ED:2 * ED, 8:16].set(AEP8)
    TGL = jnp.zeros((T, 16), f32).at[:, 0:8].set(tG8)
    TGR = jnp.zeros((T, 16), f32).at[:, 8:16].set(tG8)
    one64 = jnp.ones((1, 64), f32)
    cw = [comb_w[64 * i:64 * (i + 1)] for i in range(4)]
    cwb = comb_w[256:]
    zb = (comb_b + jnp.dot(bias_gat, comb_w[:256]))[None]
    w1a = mlp_w1[:64]
    w1b = mlp_w1[64:]

    r0, r1, r2, r3, x0, x1, asd = _tca(x, W_gat, Asd)
    etP2 = edge_types.astype(jnp.int32).reshape(E // 2, 2)
    epp0p, epp1p, ae8p = _tcb(ea32, etP2, W0p, W1p, T0L, T0R, T1L, T1R,
                              AEPp, TGL, TGR)
    q0, q1, q2, q3, g0, g1, denom_p = _sc(
        r0, r1, r2, r3, x0, x1, asd, ae8p.reshape(E * 8),
        epp0p.reshape(E * 64), epp1p.reshape(E * 64), src, dst)
    return _tcc(x, (q0, q1, q2, q3), (g0, g1), denom_p, one64, w1a, w1b,
                mlp_b1.reshape(1, GINE), mlp_w2, mlp_b2.reshape(1, GINE),
                cw, cwb, zb, ln_gamma.reshape(1, OUT), ln_beta.reshape(1, OUT))


# speedup vs baseline: 2.1709x; 1.0868x over previous
import functools

import jax
import jax.numpy as jnp
from jax import lax
from jax.experimental import pallas as pl
from jax.experimental.pallas import tpu as pltpu
from jax.experimental.pallas import tpu_sc as plsc

N = 10000
E = 160000
D = 128
ED = 16
H = 4
C = 64
T = 8
GINE = 64
OUT = 128

NC = 2
NS = 16
NW = NC * NS
EK = 128
NCHUNK = E // EK
MAXCH = -(-NCHUNK // NW)
NP = 10240
RPT = NP // NS
ZCHUNKS = ((0, 128), (128, 128), (256, 128), (384, 128), (512, 128))

_BN = 1000
_BE = 2000

def _mesh():
    return plsc.VectorSubcoreMesh(
        core_axis_name="c", subcore_axis_name="s", num_cores=NC, num_subcores=NS)


def _tca_body(x_ref, wg_ref, asdw_ref, h0_ref, h1_ref, asd_ref):
    h = jnp.dot(x_ref[...], wg_ref[...], preferred_element_type=jnp.float32)
    h0_ref[...] = h[:, :128]
    h1_ref[...] = h[:, 128:]
    asd_ref[...] = jnp.dot(h, asdw_ref[...], preferred_element_type=jnp.float32)


def _tca(x, W_gat, Asd):
    return pl.pallas_call(
        _tca_body,
        grid=(N // _BN,),
        in_specs=[
            pl.BlockSpec((_BN, D), lambda i: (i, 0)),
            pl.BlockSpec((D, H * C), lambda i: (0, 0)),
            pl.BlockSpec((H * C, 8), lambda i: (0, 0)),
        ],
        out_specs=[
            pl.BlockSpec((_BN, 128), lambda i: (i, 0)),
            pl.BlockSpec((_BN, 128), lambda i: (i, 0)),
            pl.BlockSpec((_BN, 8), lambda i: (i, 0)),
        ],
        out_shape=[
            jax.ShapeDtypeStruct((N, 128), jnp.float32),
            jax.ShapeDtypeStruct((N, 128), jnp.float32),
            jax.ShapeDtypeStruct((N, 8), jnp.float32),
        ],
    )(x, W_gat, Asd)


def _tcb_body(ea_ref, et_ref, elw_ref, tE_ref, aep_ref, tG_ref, epp_ref, ae8_ref):
    ea = ea_ref[...]
    et = et_ref[...]
    oh = (lax.broadcasted_iota(jnp.int32, (_BE, T), 1) == et).astype(jnp.float32)
    epp_ref[...] = (
        jnp.dot(ea, elw_ref[...], preferred_element_type=jnp.float32)
        + jnp.dot(oh, tE_ref[...], preferred_element_type=jnp.float32))
    ae8_ref[...] = (
        jnp.dot(ea, aep_ref[...], preferred_element_type=jnp.float32)
        + jnp.dot(oh, tG_ref[...], preferred_element_type=jnp.float32))


def _tcb(edge_attr, et2d, edge_lin_w, tE, AEP8, tG8):
    return pl.pallas_call(
        _tcb_body,
        grid=(E // _BE,),
        in_specs=[
            pl.BlockSpec((_BE, ED), lambda i: (i, 0)),
            pl.BlockSpec((_BE, 1), lambda i: (i, 0)),
            pl.BlockSpec((ED, D), lambda i: (0, 0)),
            pl.BlockSpec((T, D), lambda i: (0, 0)),
            pl.BlockSpec((ED, 8), lambda i: (0, 0)),
            pl.BlockSpec((T, 8), lambda i: (0, 0)),
        ],
        out_specs=[
            pl.BlockSpec((_BE, D), lambda i: (i, 0)),
            pl.BlockSpec((_BE, 8), lambda i: (i, 0)),
        ],
        out_shape=[
            jax.ShapeDtypeStruct((E, D), jnp.float32),
            jax.ShapeDtypeStruct((E, 8), jnp.float32),
        ],
    )(edge_attr, et2d, edge_lin_w, tE, AEP8, tG8)


def _zero_vmem_rows(ref, nrows, ncols):
    z16 = jnp.zeros((16,), jnp.float32)

    def body(r, _):
        for cb in range(ncols // 16):
            ref[r, pl.ds(cb * 16, 16)] = z16
        return 0

    lax.fori_loop(0, nrows, body, 0, unroll=False)


def _zero_my_shared_rows(zbuf, shared, base):
    for off, cnt in ZCHUNKS:
        pltpu.sync_copy(zbuf.at[pl.ds(0, cnt)], shared.at[pl.ds(base + off, cnt)])


def _dump_my_shared_rows(shared, out, cid, base):
    for off, cnt in ZCHUNKS:
        pltpu.sync_copy(shared.at[pl.ds(base + off, cnt)],
                        out.at[cid, pl.ds(base + off, cnt)])


def _scale_rows_by_heads(h_v, ex4_v, h_lo):
    def body(e, _):
        ef = jnp.full((16,), e * 4, jnp.int32)
        b0 = plsc.load_gather(ex4_v, [ef + h_lo])
        b1 = plsc.load_gather(ex4_v, [ef + (h_lo + 1)])
        for cb in range(8):
            b = b0 if cb < 4 else b1
            sl = pl.ds(cb * 16, 16)
            h_v[e, sl] = h_v[e, sl] * b
        return 0

    lax.fori_loop(0, EK, body, 0, unroll=False)


def _compute_ex(asrc_v, adst_v, ae_v, ex4_v, ex8_v, iota16):
    for j16 in range(8):
        rows = iota16 + (j16 * 16)
        for j in range(H):
            colj = jnp.full((16,), j, jnp.int32)
            a_s = plsc.load_gather(asrc_v, [rows, colj])
            a_d = plsc.load_gather(adst_v, [rows, colj + 4])
            a_e = plsc.load_gather(ae_v, [rows, colj])
            lg = a_s + a_d + a_e
            lg = jnp.maximum(lg, lg * 0.2)
            exv = jnp.exp(lg)
            plsc.store_scatter(ex4_v, [rows * 4 + j], exv)
            if ex8_v is not None:
                plsc.store_scatter(ex8_v, [rows, colj], exv)


def _sc_body(h0, h1, xH, asd, ae8, eppH, srcH, dstH,
             gat01_out, gat23_out, gine_out, denom_out,
             src_i, dst_i, ae_v, asrc_v, adst_v, ex4_v, ex8_v, h_v, epp_v,
             denom_sh, acc_sh):
    cid = lax.axis_index("c")
    sid = lax.axis_index("s")
    wid = sid * NC + cid
    base = sid * RPT

    iota16 = lax.iota(jnp.int32, 16)
    _zero_vmem_rows(h_v, 128, 128)

    def zex(g, _):
        rows = iota16 // 8 + 2 * g
        cols = jnp.bitwise_and(iota16, 7)
        plsc.store_scatter(ex8_v, [rows, cols], jnp.zeros((16,), jnp.float32))
        return 0

    lax.fori_loop(0, EK // 2, zex, 0, unroll=False)
    _zero_my_shared_rows(h_v, acc_sh, base)
    _zero_my_shared_rows(ex8_v, denom_sh, base)
    plsc.subcore_barrier()

    def chunk_a(i, _):
        c = wid + i * NW

        @pl.when(c < NCHUNK)
        def _():
            e0 = c * EK
            pltpu.sync_copy(srcH.at[pl.ds(e0, EK)], src_i)
            pltpu.sync_copy(dstH.at[pl.ds(e0, EK)], dst_i)
            pltpu.sync_copy(ae8.at[pl.ds(e0, EK)], ae_v)
            pltpu.sync_copy(asd.at[src_i], asrc_v)
            pltpu.sync_copy(asd.at[dst_i], adst_v)
            pltpu.sync_copy(h0.at[src_i], h_v)
            _compute_ex(asrc_v, adst_v, ae_v, ex4_v, ex8_v, iota16)
            _scale_rows_by_heads(h_v, ex4_v, 0)
            pltpu.sync_copy(ex8_v, denom_sh.at[dst_i], add=True)
            pltpu.sync_copy(h_v, acc_sh.at[dst_i], add=True)

        return 0

    lax.fori_loop(0, MAXCH, chunk_a, 0, unroll=False)
    plsc.subcore_barrier()
    _dump_my_shared_rows(acc_sh, gat01_out, cid, base)
    _dump_my_shared_rows(denom_sh, denom_out, cid, base)
    _zero_vmem_rows(h_v, 128, 128)
    _zero_my_shared_rows(h_v, acc_sh, base)
    plsc.subcore_barrier()

    def chunk_b(i, _):
        c = wid + i * NW

        @pl.when(c < NCHUNK)
        def _():
            e0 = c * EK
            pltpu.sync_copy(srcH.at[pl.ds(e0, EK)], src_i)
            pltpu.sync_copy(dstH.at[pl.ds(e0, EK)], dst_i)
            pltpu.sync_copy(ae8.at[pl.ds(e0, EK)], ae_v)
            pltpu.sync_copy(asd.at[src_i], asrc_v)
            pltpu.sync_copy(asd.at[dst_i], adst_v)
            pltpu.sync_copy(h1.at[src_i], h_v)
            _compute_ex(asrc_v, adst_v, ae_v, ex4_v, None, iota16)
            _scale_rows_by_heads(h_v, ex4_v, 2)
            pltpu.sync_copy(h_v, acc_sh.at[dst_i], add=True)

        return 0

    lax.fori_loop(0, MAXCH, chunk_b, 0, unroll=False)
    plsc.subcore_barrier()
    _dump_my_shared_rows(acc_sh, gat23_out, cid, base)
    _zero_vmem_rows(h_v, 128, 128)
    _zero_my_shared_rows(h_v, acc_sh, base)
    plsc.subcore_barrier()

    def chunk_c(i, _):
        c = wid + i * NW

        @pl.when(c < NCHUNK)
        def _():
            e0 = c * EK
            pltpu.sync_copy(srcH.at[pl.ds(e0, EK)], src_i)
            pltpu.sync_copy(dstH.at[pl.ds(e0, EK)], dst_i)
            pltpu.sync_copy(eppH.at[pl.ds(e0, EK)], epp_v)
            pltpu.sync_copy(xH.at[src_i], h_v)

            def body(e, _):
                for cb in range(8):
                    sl = pl.ds(cb * 16, 16)
                    v = h_v[e, sl] + epp_v[e, sl]
                    h_v[e, sl] = jnp.maximum(v, 0.0)
                return 0

            lax.fori_loop(0, EK, body, 0, unroll=False)
            pltpu.sync_copy(h_v, acc_sh.at[dst_i], add=True)

        return 0

    lax.fori_loop(0, MAXCH, chunk_c, 0, unroll=False)
    plsc.subcore_barrier()
    _dump_my_shared_rows(acc_sh, gine_out, cid, base)


def _sc(h0, h1, x, asd, ae8, epp, src, dst):
    return pl.kernel(
        _sc_body,
        out_type=[
            jax.ShapeDtypeStruct((NC, NP, 128), jnp.float32),
            jax.ShapeDtypeStruct((NC, NP, 128), jnp.float32),
            jax.ShapeDtypeStruct((NC, NP, 128), jnp.float32),
            jax.ShapeDtypeStruct((NC, NP, 8), jnp.float32),
        ],
        mesh=_mesh(),
        compiler_params=pltpu.CompilerParams(
            use_tc_tiling_on_sc=False, needs_layout_passes=False),
        scratch_types=[
            pltpu.VMEM((EK,), jnp.int32),
            pltpu.VMEM((EK,), jnp.int32),
            pltpu.VMEM((EK, 8), jnp.float32),
            pltpu.VMEM((EK, 8), jnp.float32),
            pltpu.VMEM((EK, 8), jnp.float32),
            pltpu.VMEM((EK * 4,), jnp.float32),
            pltpu.VMEM((EK, 8), jnp.float32),
            pltpu.VMEM((EK, 128), jnp.float32),
            pltpu.VMEM((EK, 128), jnp.float32),
            pltpu.VMEM_SHARED((NP, 8), jnp.float32),
            pltpu.VMEM_SHARED((NP, 128), jnp.float32),
        ],
    )(h0, h1, x, asd, ae8, epp, src, dst)


def _tcc_body(x_ref, g0_ref, g1_ref, d_ref, gi_ref, e2_ref, w1_ref, b1_ref,
              w2_ref, b2_ref, cwa0_ref, cwa1_ref, cwb_ref, zb_ref, lg_ref,
              lb_ref, out_ref):
    num0 = g0_ref[0] + g0_ref[1]
    num1 = g1_ref[0] + g1_ref[1]
    den = d_ref[0, :, :4] + d_ref[1, :, :4]
    dinv = 1.0 / (den + 1e-16)
    e2 = e2_ref[...]
    s01 = jnp.dot(dinv[:, :2], e2, preferred_element_type=jnp.float32)
    s23 = jnp.dot(dinv[:, 2:], e2, preferred_element_type=jnp.float32)
    z = (jnp.dot(num0 * s01, cwa0_ref[...], preferred_element_type=jnp.float32)
         + jnp.dot(num1 * s23, cwa1_ref[...], preferred_element_type=jnp.float32))
    hg = x_ref[...] + gi_ref[0] + gi_ref[1]
    t = jnp.maximum(
        jnp.dot(hg, w1_ref[...], preferred_element_type=jnp.float32)
        + b1_ref[...], 0.0)
    g = jnp.dot(t, w2_ref[...], preferred_element_type=jnp.float32) + b2_ref[...]
    z = z + jnp.dot(g, cwb_ref[...], preferred_element_type=jnp.float32) + zb_ref[...]
    mu = jnp.mean(z, axis=-1, keepdims=True)
    zc = z - mu
    var = jnp.mean(zc * zc, axis=-1, keepdims=True)
    zn = zc * lax.rsqrt(var + 1e-5) * lg_ref[...] + lb_ref[...]
    out_ref[...] = jnp.maximum(zn, 0.0)


def _tcc(x, gat0_p, gat1_p, denom_p, gine_p, E2, mlp_w1, mlp_b1, mlp_w2,
         mlp_b2, cwa0, cwa1, cwb, zb, ln_gamma, ln_beta):
    full = lambda *shape: pl.BlockSpec(shape, lambda i: (0,) * len(shape))
    return pl.pallas_call(
        _tcc_body,
        grid=(N // _BN,),
        in_specs=[
            pl.BlockSpec((_BN, D), lambda i: (i, 0)),
            pl.BlockSpec((NC, _BN, 128), lambda i: (0, i, 0)),
            pl.BlockSpec((NC, _BN, 128), lambda i: (0, i, 0)),
            pl.BlockSpec((NC, _BN, 8), lambda i: (0, i, 0)),
            pl.BlockSpec((NC, _BN, 128), lambda i: (0, i, 0)),
            full(2, 128),
            full(D, GINE),
            full(1, GINE),
            full(GINE, GINE),
            full(1, GINE),
            full(128, OUT),
            full(128, OUT),
            full(GINE, OUT),
            full(1, OUT),
            full(1, OUT),
            full(1, OUT),
        ],
        out_specs=pl.BlockSpec((_BN, OUT), lambda i: (i, 0)),
        out_shape=jax.ShapeDtypeStruct((N, OUT), jnp.float32),
    )(x, gat0_p, gat1_p, denom_p, gine_p, E2, mlp_w1, mlp_b1, mlp_w2, mlp_b2,
      cwa0, cwa1, cwb, zb, ln_gamma, ln_beta)


def kernel(x, edge_index, edge_attr, edge_types, type_emb_gat, W_gat,
           W_edge_gat, att_src, att_dst, att_edge, bias_gat, type_emb_gine,
           edge_lin_w, edge_lin_b, mlp_w1, mlp_b1, mlp_w2, mlp_b2, comb_w,
           comb_b, ln_gamma, ln_beta):
    src = edge_index[0].astype(jnp.int32)
    dst = edge_index[1].astype(jnp.int32)
    et2d = edge_types.astype(jnp.int32).reshape(E, 1)

    ar = jnp.arange(H)
    Asrc = jnp.zeros((H, C, H), jnp.float32).at[ar, :, ar].set(att_src)
    Adst = jnp.zeros((H, C, H), jnp.float32).at[ar, :, ar].set(att_dst)
    Asd = jnp.concatenate(
        [Asrc.reshape(H * C, H), Adst.reshape(H * C, H)], axis=1)
    AEP = jnp.einsum("ehc,hc->eh", W_edge_gat.reshape(ED, H, C), att_edge)
    AEP8 = jnp.pad(AEP, ((0, 0), (0, 4)))
    tG8 = jnp.dot(type_emb_gat, AEP8)
    tE = jnp.dot(type_emb_gine, edge_lin_w) + edge_lin_b[None]
    E2 = jnp.repeat(jnp.eye(2, dtype=jnp.float32), 64, axis=1)
    cwa0 = comb_w[:128]
    cwa1 = comb_w[128:256]
    cwb = comb_w[256:]
    zb = (comb_b + jnp.dot(bias_gat, comb_w[:256]))[None]

    h0, h1, asd = _tca(x, W_gat, Asd)
    epp, ae8 = _tcb(edge_attr, et2d, edge_lin_w, tE, AEP8, tG8)
    gat0_p, gat1_p, gine_p, denom_p = _sc(h0, h1, x, asd, ae8, epp, src, dst)
    return _tcc(x, gat0_p, gat1_p, denom_p, gine_p, E2, mlp_w1,
                mlp_b1.reshape(1, GINE), mlp_w2, mlp_b2.reshape(1, GINE),
                cwa0, cwa1, cwb, zb, ln_gamma.reshape(1, OUT),
                ln_beta.reshape(1, OUT))
